# baseline (speedup 1.0000x reference)
import functools
import math

import jax
import jax.numpy as jnp
from jax import lax
from jax.experimental import pallas as pl
from jax.experimental.pallas import tpu as pltpu
from jax.experimental.pallas import tpu_sc as plsc

VOCAB = 1000000
EMBED = 64
EPAD = 128
BATCH = 4096
SEQ = 200
B = BATCH * SEQ

L = 16
NC, NS = 2, 16
NW = NC * NS
SCALE = math.sqrt(EMBED)

NBLK = VOCAB // EPAD
VTAIL = NBLK * EPAD
NTAIL = VOCAB - VTAIL
TRIPS = (NBLK + NW - 1) // NW

B_PER_W = B // NW
SUB = 128
CHUNK = 256
NSUB = CHUNK // SUB
NCHUNK = B_PER_W // CHUNK
IDXROWS_PER_W = B_PER_W // SUB


def _prep_kernel(tt_hbm, ttail_hbm, tpad_hbm,
                 tin0, tin1, tout0, tout1, tailv, gsem0, gsem1, ssem0, ssem1):
    wid = lax.axis_index("s") * NC + lax.axis_index("c")
    tin = (tin0, tin1)
    tout = (tout0, tout1)
    gsem = (gsem0, gsem1)
    ssem = (ssem0, ssem1)
    iota = lax.iota(jnp.int32, L)

    def fire(b, m):
        pltpu.async_copy(tt_hbm.at[:, pl.ds(m * EPAD, EPAD)], tin[b], gsem[b])

    def transpose(b):
        src = tin[b]
        dst = tout[b]

        @plsc.parallel_loop(0, EPAD, step=1, unroll=2)
        def _(l):
            lane = jnp.full((L,), 0, jnp.int32) + l
            for j in range(EMBED // L):
                col = plsc.load_gather(src, [iota + (j * L), lane])
                dst[l, pl.ds(j * L, L)] = col * SCALE

    fire(0, wid)

    def trip_body(t, carry):
        b = lax.rem(t, 2)
        m = t * NW + wid
        for bb in range(2):

            @pl.when(b == bb)
            def _():
                pltpu.make_async_copy(
                    tt_hbm.at[:, pl.ds(m * EPAD, EPAD)], tin[bb], gsem[bb]).wait()

                nm = m + NW

                @pl.when(nm < NBLK)
                def _():
                    fire(1 - bb, nm)

                @pl.when(t >= 2)
                def _():
                    pltpu.make_async_copy(
                        tout[bb],
                        tpad_hbm.at[pl.ds((m - 2 * NW) * EPAD, EPAD)],
                        ssem[bb]).wait()

                transpose(bb)
                pltpu.async_copy(
                    tout[bb], tpad_hbm.at[pl.ds(m * EPAD, EPAD)], ssem[bb])
        return carry

    full_trips = NBLK // NW
    lax.fori_loop(0, full_trips, trip_body, 0)

    has_extra = full_trips * NW + wid < NBLK

    @pl.when(has_extra)
    def _():
        trip_body(jnp.int32(full_trips), 0)

    for bb in range(2):
        cand_t = TRIPS - 1 - (TRIPS - 1 - bb) % 2
        last_t = jnp.where(
            jnp.logical_or(has_extra, cand_t < full_trips),
            jnp.int32(cand_t), jnp.int32(cand_t - 2))
        pltpu.make_async_copy(
            tout[bb],
            tpad_hbm.at[pl.ds((last_t * NW + wid) * EPAD, EPAD)],
            ssem[bb]).wait()

    @pl.when(wid == 0)
    def _():
        pltpu.sync_copy(ttail_hbm, tailv)

        @plsc.parallel_loop(0, NTAIL, step=1, unroll=2)
        def _(r):
            for j in range(EMBED // L):
                tout0[r, pl.ds(j * L, L)] = tailv[r, pl.ds(j * L, L)] * SCALE

        pltpu.sync_copy(
            tout0.at[pl.ds(0, NTAIL)], tpad_hbm.at[pl.ds(VTAIL, NTAIL)])


def _emb_kernel(idx_hbm, tab_hbm, out_hbm,
                idx0, idx1, rows0, rows1, cmp_v, sem0, sem1):
    wid = lax.axis_index("s") * NC + lax.axis_index("c")
    out_base = wid * B_PER_W
    idx_base = wid * IDXROWS_PER_W
    idx_v = (idx0, idx1)
    rows_v = (rows0, rows1)
    sems = (sem0, sem1)

    def fire(b, c):
        pltpu.sync_copy(idx_hbm.at[pl.ds(idx_base + c * NSUB, NSUB)], idx_v[b])
        for j in range(NSUB):
            pltpu.async_copy(
                tab_hbm.at[idx_v[b].at[j]],
                rows_v[b].at[pl.ds(j * SUB, SUB)],
                sems[b],
            )

    def drain(b):
        for j in range(NSUB):
            pltpu.make_async_copy(
                tab_hbm.at[idx_v[b].at[j]],
                rows_v[b].at[pl.ds(j * SUB, SUB)],
                sems[b],
            ).wait()

    def compact(b):
        rows = rows_v[b]

        @plsc.parallel_loop(0, CHUNK, step=1, unroll=8)
        def _(i):
            for j in range(EMBED // L):
                cmp_v[i, pl.ds(j * L, L)] = rows[i, pl.ds(j * L, L)]

    for b in range(2):
        fire(b, b)

    def group_body(g, carry):
        for b in range(2):
            c = g * 2 + b
            drain(b)
            compact(b)

            @pl.when(c + 2 < NCHUNK)
            def _():
                fire(b, c + 2)

            pltpu.sync_copy(cmp_v, out_hbm.at[pl.ds(out_base + c * CHUNK, CHUNK)])
        return carry

    lax.fori_loop(0, NCHUNK // 2, group_body, 0)


@jax.jit
def _run(idx2d, tt, ttail):
    mesh = plsc.VectorSubcoreMesh(core_axis_name="c", subcore_axis_name="s")
    params = pltpu.CompilerParams(
        use_tc_tiling_on_sc=True, needs_layout_passes=False)
    tpad = pl.kernel(
        _prep_kernel,
        mesh=mesh,
        out_type=jax.ShapeDtypeStruct((VOCAB, EPAD), jnp.float32),
        scratch_types=[
            pltpu.VMEM((EMBED, EPAD), jnp.float32),
            pltpu.VMEM((EMBED, EPAD), jnp.float32),
            pltpu.VMEM((EPAD, EPAD), jnp.float32),
            pltpu.VMEM((EPAD, EPAD), jnp.float32),
            pltpu.VMEM((NTAIL, EMBED), jnp.float32),
            pltpu.SemaphoreType.DMA,
            pltpu.SemaphoreType.DMA,
            pltpu.SemaphoreType.DMA,
            pltpu.SemaphoreType.DMA,
        ],
        compiler_params=params,
    )(tt, ttail)
    return pl.kernel(
        _emb_kernel,
        mesh=mesh,
        out_type=jax.ShapeDtypeStruct((B, EMBED), jnp.float32),
        scratch_types=[
            pltpu.VMEM((NSUB, SUB), jnp.int32),
            pltpu.VMEM((NSUB, SUB), jnp.int32),
            pltpu.VMEM((CHUNK, EPAD), jnp.float32),
            pltpu.VMEM((CHUNK, EPAD), jnp.float32),
            pltpu.VMEM((CHUNK, EMBED), jnp.float32),
            pltpu.SemaphoreType.DMA,
            pltpu.SemaphoreType.DMA,
        ],
        compiler_params=params,
    )(idx2d, tpad)


def kernel(inputs, table):
    idx2d = inputs.reshape(B // SUB, SUB)
    tt = table.T
    ttail = lax.slice(table, (VTAIL, 0), (VOCAB, EMBED))
    out = _run(idx2d, tt, ttail)
    return out.reshape(BATCH, SEQ, EMBED)


# speedup vs baseline: 1.2557x; 1.2557x over previous
import functools
import math

import jax
import jax.numpy as jnp
from jax import lax
from jax.experimental import pallas as pl
from jax.experimental.pallas import tpu as pltpu
from jax.experimental.pallas import tpu_sc as plsc

VOCAB = 1000000
EMBED = 64
EPAD = 128
BATCH = 4096
SEQ = 200
B = BATCH * SEQ

L = 16
NC, NS = 2, 16
NW = NC * NS
B_PER_W = B // NW
SUB = 128
CHUNK = 256
HALF = CHUNK // 2
NSUB = CHUNK // SUB
NBUF = 3
NCHUNK = B_PER_W // CHUNK
IDXROWS_PER_W = B_PER_W // SUB
SCALE = math.sqrt(EMBED)


def _emb_kernel(idx_hbm, tab_hbm, out_hbm,
                idx0, idx1, idx2, rows0, rows1, rows2, cmp_v,
                sem0, sem1, sem2):
    wid = lax.axis_index("s") * NC + lax.axis_index("c")
    out_base = wid * B_PER_W
    idx_base = wid * IDXROWS_PER_W
    idx_v = (idx0, idx1, idx2)
    rows_v = (rows0, rows1, rows2)
    sems = (sem0, sem1, sem2)

    def fire(b, c):
        pltpu.sync_copy(idx_hbm.at[pl.ds(idx_base + c * NSUB, NSUB)], idx_v[b])
        for j in range(NSUB):
            pltpu.async_copy(
                tab_hbm.at[idx_v[b].at[j]],
                rows_v[b].at[pl.ds(j * SUB, SUB)],
                sems[b],
            )

    def drain(b):
        for j in range(NSUB):
            pltpu.make_async_copy(
                tab_hbm.at[idx_v[b].at[j]],
                rows_v[b].at[pl.ds(j * SUB, SUB)],
                sems[b],
            ).wait()

    def emit(b, c):
        rows = rows_v[b]
        drain(b)
        for h in range(2):

            @plsc.parallel_loop(0, HALF, step=1, unroll=8)
            def _(i):
                for j in range(EMBED // L):
                    cmp_v[i, pl.ds(j * L, L)] = (
                        rows[h * HALF + i, pl.ds(j * L, L)] * SCALE)

            if h == 1:

                @pl.when(c + NBUF < NCHUNK)
                def _():
                    fire(b, c + NBUF)

            pltpu.sync_copy(
                cmp_v,
                out_hbm.at[pl.ds(out_base + c * CHUNK + h * HALF, HALF)])

    for b in range(NBUF):
        fire(b, b)

    def group_body(g, carry):
        for b in range(NBUF):
            emit(b, g * NBUF + b)
        return carry

    lax.fori_loop(0, NCHUNK // NBUF, group_body, 0)
    for c in range(NCHUNK - NCHUNK % NBUF, NCHUNK):
        emit(c % NBUF, jnp.int32(c))


@jax.jit
def _emb(idx2d, tpad):
    mesh = plsc.VectorSubcoreMesh(core_axis_name="c", subcore_axis_name="s")
    return pl.kernel(
        _emb_kernel,
        mesh=mesh,
        out_type=jax.ShapeDtypeStruct((B, EMBED), jnp.float32),
        scratch_types=[
            pltpu.VMEM((NSUB, SUB), jnp.int32),
            pltpu.VMEM((NSUB, SUB), jnp.int32),
            pltpu.VMEM((NSUB, SUB), jnp.int32),
            pltpu.VMEM((CHUNK, EPAD), jnp.float32),
            pltpu.VMEM((CHUNK, EPAD), jnp.float32),
            pltpu.VMEM((CHUNK, EPAD), jnp.float32),
            pltpu.VMEM((HALF, EMBED), jnp.float32),
            pltpu.SemaphoreType.DMA,
            pltpu.SemaphoreType.DMA,
            pltpu.SemaphoreType.DMA,
        ],
        compiler_params=pltpu.CompilerParams(use_tc_tiling_on_sc=True),
    )(idx2d, tpad)


def kernel(inputs, table):
    idx2d = inputs.reshape(B // SUB, SUB)
    tpad = jnp.pad(table, ((0, 0), (0, EPAD - EMBED)))
    out = _emb(idx2d, tpad)
    return out.reshape(BATCH, SEQ, EMBED)


# speedup vs baseline: 1.2752x; 1.0155x over previous
import functools
import math

import jax
import jax.numpy as jnp
from jax import lax
from jax.experimental import pallas as pl
from jax.experimental.pallas import tpu as pltpu
from jax.experimental.pallas import tpu_sc as plsc

VOCAB = 1000000
EMBED = 64
EPAD = 128
BATCH = 4096
SEQ = 200
B = BATCH * SEQ

L = 16
NC, NS = 2, 16
NW = NC * NS
B_PER_W = B // NW
SUB = 128
CHUNK = 256
HALF = CHUNK // 2
NSUB = CHUNK // SUB
NBUF = 3
NCHUNK = B_PER_W // CHUNK
IDXROWS_PER_W = B_PER_W // SUB
SCALE = math.sqrt(EMBED)


def _emb_kernel(idx_hbm, tab_hbm, out_hbm,
                idx0, idx1, idx2, rows0, rows1, rows2, cmp_v,
                sem0, sem1, sem2, isem0, isem1, isem2):
    wid = lax.axis_index("s") * NC + lax.axis_index("c")
    out_base = wid * B_PER_W
    idx_base = wid * IDXROWS_PER_W
    idx_v = (idx0, idx1, idx2)
    rows_v = (rows0, rows1, rows2)
    sems = (sem0, sem1, sem2)
    isems = (isem0, isem1, isem2)

    def prefetch_idx(b, c):
        pltpu.async_copy(
            idx_hbm.at[pl.ds(idx_base + c * NSUB, NSUB)], idx_v[b], isems[b])

    def fire(b, c):
        pltpu.make_async_copy(
            idx_hbm.at[pl.ds(idx_base + c * NSUB, NSUB)], idx_v[b],
            isems[b]).wait()
        for j in range(NSUB):
            pltpu.async_copy(
                tab_hbm.at[idx_v[b].at[j]],
                rows_v[b].at[pl.ds(j * SUB, SUB)],
                sems[b],
            )

    def drain(b):
        for j in range(NSUB):
            pltpu.make_async_copy(
                tab_hbm.at[idx_v[b].at[j]],
                rows_v[b].at[pl.ds(j * SUB, SUB)],
                sems[b],
            ).wait()

    def emit(b, c):
        rows = rows_v[b]
        drain(b)

        @pl.when(c + NBUF < NCHUNK)
        def _():
            prefetch_idx(b, c + NBUF)

        for h in range(2):

            @plsc.parallel_loop(0, HALF, step=1, unroll=8)
            def _(i):
                for j in range(EMBED // L):
                    cmp_v[i, pl.ds(j * L, L)] = (
                        rows[h * HALF + i, pl.ds(j * L, L)] * SCALE)

            if h == 1:

                @pl.when(c + NBUF < NCHUNK)
                def _():
                    fire(b, c + NBUF)

            pltpu.sync_copy(
                cmp_v,
                out_hbm.at[pl.ds(out_base + c * CHUNK + h * HALF, HALF)])

    for b in range(NBUF):
        prefetch_idx(b, b)
        fire(b, b)

    def group_body(g, carry):
        for b in range(NBUF):
            emit(b, g * NBUF + b)
        return carry

    lax.fori_loop(0, NCHUNK // NBUF, group_body, 0)
    for c in range(NCHUNK - NCHUNK % NBUF, NCHUNK):
        emit(c % NBUF, jnp.int32(c))


@jax.jit
def _emb(idx2d, tpad):
    mesh = plsc.VectorSubcoreMesh(core_axis_name="c", subcore_axis_name="s")
    return pl.kernel(
        _emb_kernel,
        mesh=mesh,
        out_type=jax.ShapeDtypeStruct((B, EMBED), jnp.float32),
        scratch_types=[
            pltpu.VMEM((NSUB, SUB), jnp.int32),
            pltpu.VMEM((NSUB, SUB), jnp.int32),
            pltpu.VMEM((NSUB, SUB), jnp.int32),
            pltpu.VMEM((CHUNK, EPAD), jnp.float32),
            pltpu.VMEM((CHUNK, EPAD), jnp.float32),
            pltpu.VMEM((CHUNK, EPAD), jnp.float32),
            pltpu.VMEM((HALF, EMBED), jnp.float32),
            pltpu.SemaphoreType.DMA,
            pltpu.SemaphoreType.DMA,
            pltpu.SemaphoreType.DMA,
            pltpu.SemaphoreType.DMA,
            pltpu.SemaphoreType.DMA,
            pltpu.SemaphoreType.DMA,
        ],
        compiler_params=pltpu.CompilerParams(use_tc_tiling_on_sc=True),
    )(idx2d, tpad)


def kernel(inputs, table):
    idx2d = inputs.reshape(B // SUB, SUB)
    tpad = jnp.pad(table, ((0, 0), (0, EPAD - EMBED)))
    out = _emb(idx2d, tpad)
    return out.reshape(BATCH, SEQ, EMBED)
